# baseline (speedup 1.0000x reference)
import functools

import jax
import jax.numpy as jnp
from jax import lax
from jax.experimental import pallas as pl
from jax.experimental.pallas import tpu as pltpu
from jax.experimental.pallas import tpu_sc as plsc

B = 16384
D = 16

_info = plsc.get_sparse_core_info()
NC = _info.num_cores
NS = _info.num_subcores
L = _info.num_lanes
NW = NC * NS
BPW = B // NW
G = BPW // L

_mesh = plsc.VectorSubcoreMesh(core_axis_name="c", subcore_axis_name="s")


@functools.partial(
    pl.kernel,
    mesh=_mesh,
    compiler_params=pltpu.CompilerParams(
        needs_layout_passes=False, use_tc_tiling_on_sc=True),
    out_type=jax.ShapeDtypeStruct((B,), jnp.float32),
    scratch_types=[
        pltpu.VMEM((BPW,), jnp.int32),
        pltpu.VMEM((BPW,), jnp.int32),
        pltpu.VMEM((8, NC, 8, L * L), jnp.float32),
        pltpu.VMEM((8, NC, 8, L * L), jnp.float32),
        pltpu.VMEM((BPW,), jnp.float32),
        pltpu.SemaphoreType.DMA,
    ],
)
def _mf_kernel(uidx_hbm, iidx_hbm, utab_hbm, itab_hbm, out_hbm,
               uidx_v, iidx_v, uchunks, ichunks, outv, sem):
    wid = lax.axis_index("s") * NC + lax.axis_index("c")

    pltpu.sync_copy(uidx_hbm.at[wid], uidx_v)
    pltpu.sync_copy(iidx_hbm.at[wid], iidx_v)

    lanes = lax.iota(jnp.int32, L)

    def fire(g):
        slot = g & 7
        uvec = uidx_v[pl.ds(g * L, L)]
        ivec = iidx_v[pl.ds(g * L, L)]
        ubase = uvec & jnp.int32(~(L - 1))
        ibase = ivec & jnp.int32(~(L - 1))
        for l in range(L):
            c_u = pl.multiple_of(
                jnp.squeeze(lax.slice(ubase, (l,), (l + 1,))), L)
            c_i = pl.multiple_of(
                jnp.squeeze(lax.slice(ibase, (l,), (l + 1,))), L)
            pltpu.async_copy(
                utab_hbm.at[:, :, pl.ds(c_u, L)],
                uchunks.at[slot, :, :, pl.ds(l * L, L)], sem)
            pltpu.async_copy(
                itab_hbm.at[:, :, pl.ds(c_i, L)],
                ichunks.at[slot, :, :, pl.ds(l * L, L)], sem)

    def drain():
        pltpu.make_async_copy(
            utab_hbm.at[:, :, pl.ds(0, L * L)], uchunks.at[0], sem).wait()
        pltpu.make_async_copy(
            itab_hbm.at[:, :, pl.ds(0, L * L)], ichunks.at[0], sem).wait()

    def compute(g):
        slot = g & 7
        slotv = jnp.full((L,), slot, jnp.int32)
        uvec = uidx_v[pl.ds(g * L, L)]
        ivec = iidx_v[pl.ds(g * L, L)]
        uoff = lanes * L + (uvec & jnp.int32(L - 1))
        ioff = lanes * L + (ivec & jnp.int32(L - 1))
        acc = jnp.zeros((L,), jnp.float32)
        for a in range(NC):
            av = jnp.full((L,), a, jnp.int32)
            for b in range(8):
                bv = jnp.full((L,), b, jnp.int32)
                uval = plsc.load_gather(uchunks, [slotv, av, bv, uoff])
                ival = plsc.load_gather(ichunks, [slotv, av, bv, ioff])
                acc = acc + uval * ival
        outv[pl.ds(g * L, L)] = acc

    fire(0)
    fire(1)
    fire(2)
    fire(3)

    def step(g, carry):
        @pl.when(g + 4 < G)
        def _():
            fire(g + 4)
        drain()
        compute(g)
        return carry

    lax.fori_loop(0, G, step, 0)

    pltpu.sync_copy(outv, out_hbm.at[pl.ds(wid * BPW, BPW)])


def kernel(user_indices, item_indices, user_table, item_table):
    uidx = user_indices.astype(jnp.int32).reshape(NW, BPW)
    iidx = item_indices.astype(jnp.int32).reshape(NW, BPW)
    ut = user_table.T.reshape(2, 8, user_table.shape[0])
    it = item_table.T.reshape(2, 8, item_table.shape[0])
    return _mf_kernel(uidx, iidx, ut, it)


# speedup vs baseline: 1.0702x; 1.0702x over previous
import functools

import jax
import jax.numpy as jnp
from jax import lax
from jax.experimental import pallas as pl
from jax.experimental.pallas import tpu as pltpu
from jax.experimental.pallas import tpu_sc as plsc

B = 16384
D = 16

_info = plsc.get_sparse_core_info()
NC = _info.num_cores
NS = _info.num_subcores
L = _info.num_lanes
NW = NC * NS
BPW = B // NW
G = BPW // L

_mesh = plsc.VectorSubcoreMesh(core_axis_name="c", subcore_axis_name="s")


@functools.partial(
    pl.kernel,
    mesh=_mesh,
    compiler_params=pltpu.CompilerParams(
        needs_layout_passes=False, use_tc_tiling_on_sc=True),
    out_type=jax.ShapeDtypeStruct((B,), jnp.float32),
    scratch_types=[
        pltpu.VMEM((BPW,), jnp.int32),
        pltpu.VMEM((BPW,), jnp.int32),
        pltpu.VMEM((4, NC, 8, L * L), jnp.float32),
        pltpu.VMEM((4, NC, 8, L * L), jnp.float32),
        pltpu.VMEM((BPW,), jnp.float32),
        pltpu.SemaphoreType.DMA,
    ],
)
def _mf_kernel(uidx_hbm, iidx_hbm, utab_hbm, itab_hbm, out_hbm,
               uidx_v, iidx_v, uchunks, ichunks, outv, sem):
    wid = lax.axis_index("s") * NC + lax.axis_index("c")

    base = wid * BPW
    pltpu.sync_copy(uidx_hbm.at[pl.ds(base, BPW)], uidx_v)
    pltpu.sync_copy(iidx_hbm.at[pl.ds(base, BPW)], iidx_v)

    lanes = lax.iota(jnp.int32, L)

    def fire(g):
        slot = g & 3
        uvec = uidx_v[pl.ds(g * L, L)]
        ivec = iidx_v[pl.ds(g * L, L)]
        ubase = uvec & jnp.int32(~(L - 1))
        ibase = ivec & jnp.int32(~(L - 1))
        for l in range(L):
            c_u = pl.multiple_of(
                jnp.squeeze(lax.slice(ubase, (l,), (l + 1,))), L)
            c_i = pl.multiple_of(
                jnp.squeeze(lax.slice(ibase, (l,), (l + 1,))), L)
            pltpu.async_copy(
                utab_hbm.at[:, :, pl.ds(c_u, L)],
                uchunks.at[slot, :, :, pl.ds(l * L, L)], sem)
            pltpu.async_copy(
                itab_hbm.at[:, :, pl.ds(c_i, L)],
                ichunks.at[slot, :, :, pl.ds(l * L, L)], sem)

    def drain():
        pltpu.make_async_copy(
            utab_hbm.at[:, :, pl.ds(0, L * L)], uchunks.at[0], sem).wait()
        pltpu.make_async_copy(
            itab_hbm.at[:, :, pl.ds(0, L * L)], ichunks.at[0], sem).wait()

    def compute(g):
        slot = g & 3
        slotv = jnp.full((L,), slot, jnp.int32)
        uvec = uidx_v[pl.ds(g * L, L)]
        ivec = iidx_v[pl.ds(g * L, L)]
        uoff = lanes * L + (uvec & jnp.int32(L - 1))
        ioff = lanes * L + (ivec & jnp.int32(L - 1))
        acc = jnp.zeros((L,), jnp.float32)
        for a in range(NC):
            av = jnp.full((L,), a, jnp.int32)
            for b in range(8):
                bv = jnp.full((L,), b, jnp.int32)
                uval = plsc.load_gather(uchunks, [slotv, av, bv, uoff])
                ival = plsc.load_gather(ichunks, [slotv, av, bv, ioff])
                acc = acc + uval * ival
        outv[pl.ds(g * L, L)] = acc

    fire(0)
    fire(1)

    def step(g, carry):
        @pl.when(g + 2 < G)
        def _():
            fire(g + 2)
        drain()
        compute(g)
        return carry

    lax.fori_loop(0, G, step, 0)

    pltpu.sync_copy(outv, out_hbm.at[pl.ds(wid * BPW, BPW)])


def kernel(user_indices, item_indices, user_table, item_table):
    uidx = user_indices.astype(jnp.int32)
    iidx = item_indices.astype(jnp.int32)
    ut = user_table.T.reshape(2, 8, user_table.shape[0])
    it = item_table.T.reshape(2, 8, item_table.shape[0])
    return _mf_kernel(uidx, iidx, ut, it)
